# baseline (speedup 1.0000x reference)
import functools
import math

import jax
import jax.numpy as jnp
from jax import lax
from jax.experimental import pallas as pl
from jax.experimental.pallas import tpu as pltpu
from jax.experimental.pallas import tpu_sc as plsc

D_MODEL = 1024
SCALE = math.sqrt(D_MODEL)

NC = 2
NS = 16
LANES = 16
NW = NC * NS

CHUNK = 32
NBUF = 3
PREFETCH = 2


def _embed_lookup(x, table):
    R, S = x.shape
    D = table.shape[1]
    b_per_w = (R * S) // NW
    w_per_row = S // b_per_w
    nstep = b_per_w // CHUNK
    mesh = plsc.VectorSubcoreMesh(core_axis_name="c", subcore_axis_name="s")

    @functools.partial(
        pl.kernel,
        mesh=mesh,
        out_type=jax.ShapeDtypeStruct((R, S, D), jnp.float32),
        scratch_types=(
            [pltpu.VMEM((b_per_w,), jnp.int32)]
            + [pltpu.VMEM((CHUNK, D), jnp.float32) for _ in range(NBUF)]
            + [pltpu.SemaphoreType.DMA for _ in range(2 * NBUF)]
        ),
    )
    def k(x_hbm, table_hbm, out_hbm, idx_v, *bufs_and_sems):
        rows = bufs_and_sems[:NBUF]
        sem_in = bufs_and_sems[NBUF:2 * NBUF]
        sem_out = bufs_and_sems[2 * NBUF:]
        cid = lax.axis_index("c")
        sid = lax.axis_index("s")
        wid = sid * NC + cid
        brow = wid // w_per_row
        bcol = (wid % w_per_row) * b_per_w
        pltpu.sync_copy(x_hbm.at[brow, pl.ds(bcol, b_per_w)], idx_v)

        def gather(g, b):
            return pltpu.make_async_copy(
                table_hbm.at[idx_v.at[pl.ds(g * CHUNK, CHUNK)]],
                rows[b], sem_in[b])

        def store(g, b):
            return pltpu.make_async_copy(
                rows[b], out_hbm.at[brow, pl.ds(bcol + g * CHUNK, CHUNK)],
                sem_out[b])

        def scale(b):
            def row(r, c):
                for j in range(D // LANES):
                    sl = pl.ds(j * LANES, LANES)
                    rows[b][r, sl] = rows[b][r, sl] * SCALE
                return c
            lax.fori_loop(0, CHUNK, row, 0)

        def body(g, b, prefetch):
            gather(g, b).wait()
            if prefetch:
                gp = g + PREFETCH
                bp = (b + PREFETCH) % NBUF

                @pl.when(gp < nstep)
                def _():
                    @pl.when(gp >= NBUF)
                    def _():
                        store(gp - NBUF, bp).wait()
                    gather(gp, bp).start()
            scale(b)
            store(g, b).start()

        for g0 in range(PREFETCH):
            gather(g0, g0 % NBUF).start()

        n_main = (nstep // NBUF) * NBUF

        def outer(gg, carry):
            for b in range(NBUF):
                body(gg * NBUF + b, b, prefetch=True)
            return carry

        lax.fori_loop(0, n_main // NBUF, outer, 0)
        for g in range(n_main, nstep):
            body(g, g % NBUF, prefetch=False)
        for g in range(nstep - NBUF, nstep):
            store(g, g % NBUF).wait()

    return k(x, table)


def kernel(x, table):
    return _embed_lookup(x.astype(jnp.int32), table)


# speedup vs baseline: 1.1721x; 1.1721x over previous
import functools
import math

import jax
import jax.numpy as jnp
from jax import lax
from jax.experimental import pallas as pl
from jax.experimental.pallas import tpu as pltpu
from jax.experimental.pallas import tpu_sc as plsc

D_MODEL = 1024
SCALE = math.sqrt(D_MODEL)

NC = 2
NS = 16
LANES = 16
NW = NC * NS

CHUNK = 32
NBUF = 3
PREFETCH = 2


def _embed_lookup(x, table):
    R, S = x.shape
    D = table.shape[1]
    b_per_w = (R * S) // NW
    w_per_row = S // b_per_w
    nstep = b_per_w // CHUNK
    mesh = plsc.VectorSubcoreMesh(core_axis_name="c", subcore_axis_name="s")

    @functools.partial(
        pl.kernel,
        mesh=mesh,
        out_type=jax.ShapeDtypeStruct((R, S, D), jnp.float32),
        scratch_types=(
            [pltpu.VMEM((b_per_w,), jnp.int32)]
            + [pltpu.VMEM((CHUNK, D), jnp.float32) for _ in range(NBUF)]
            + [pltpu.SemaphoreType.DMA for _ in range(2 * NBUF)]
        ),
    )
    def k(x_hbm, table_hbm, out_hbm, idx_v, *bufs_and_sems):
        rows = bufs_and_sems[:NBUF]
        sem_in = bufs_and_sems[NBUF:2 * NBUF]
        sem_out = bufs_and_sems[2 * NBUF:]
        cid = lax.axis_index("c")
        sid = lax.axis_index("s")
        wid = sid * NC + cid
        brow = wid // w_per_row
        bcol = (wid % w_per_row) * b_per_w
        pltpu.sync_copy(x_hbm.at[brow, pl.ds(bcol, b_per_w)], idx_v)

        def gather(g, b):
            return pltpu.make_async_copy(
                table_hbm.at[idx_v.at[pl.ds(g * CHUNK, CHUNK)]],
                rows[b], sem_in[b])

        def store(g, b):
            return pltpu.make_async_copy(
                rows[b], out_hbm.at[brow, pl.ds(bcol + g * CHUNK, CHUNK)],
                sem_out[b])

        def scale(b):
            def row(r, c):
                for j in range(D // LANES):
                    sl = pl.ds(j * LANES, LANES)
                    rows[b][r, sl] = rows[b][r, sl] * SCALE
                return c
            lax.fori_loop(0, CHUNK, row, 0)

        def body(g, b, prefetch):
            gather(g, b).wait()
            scale(b)
            store(g, b).start()
            if prefetch:
                gp = g + PREFETCH
                bp = (b + PREFETCH) % NBUF

                @pl.when(gp < nstep)
                def _():
                    @pl.when(gp >= NBUF)
                    def _():
                        store(gp - NBUF, bp).wait()
                    gather(gp, bp).start()

        for g0 in range(PREFETCH):
            gather(g0, g0 % NBUF).start()

        n_main = (nstep // NBUF) * NBUF

        def outer(gg, carry):
            for b in range(NBUF):
                body(gg * NBUF + b, b, prefetch=True)
            return carry

        lax.fori_loop(0, n_main // NBUF, outer, 0)
        for g in range(n_main, nstep):
            body(g, g % NBUF, prefetch=False)
        for g in range(nstep - NBUF, nstep):
            store(g, g % NBUF).wait()

    return k(x, table)


def kernel(x, table):
    return _embed_lookup(x.astype(jnp.int32), table)
